# baseline (speedup 1.0000x reference)
import jax
import jax.numpy as jnp
from jax import lax
from jax.experimental import pallas as pl
from jax.experimental.pallas import tpu as pltpu
from jax.experimental.pallas import tpu_sc as plsc

N_POINTS = 1_000_000
NP_PAD = 1_000_064
GRID = 1024
D = 32
K = 4
B = 65536

NC = 2
NS = 16
L = 16
NW = NC * NS
PER_W = B // NW
CHUNK = 128
NCH = PER_W // CHUNK
GPC = CHUNK // L
QBLK = PER_W // 128


def _rsqrt(x):
    i = lax.bitcast_convert_type(x, jnp.int32)
    i = jnp.int32(0x5F3759DF) - lax.shift_right_arithmetic(i, 1)
    y = lax.bitcast_convert_type(i, jnp.float32)
    for _ in range(3):
        y = y * (1.5 - 0.5 * x * y * y)
    return y


def _body(pos_hbm, pts_hbm, emb_hbm, nm0_hbm, nm1_hbm, nm2_hbm, nm3_hbm,
          out_hbm,
          pos_v, qx, qy, nmidx, nidx, pxi, pyi, ptx, pty, emb_rows,
          wn, out_v, sem_nm, sem_pt0, sem_pt1, sem_emb0, sem_emb1):
    nm_hbms = (nm0_hbm, nm1_hbm, nm2_hbm, nm3_hbm)
    wid = lax.axis_index("s") * NC + lax.axis_index("c")
    base = wid * PER_W

    iota = lax.iota(jnp.int32, L)

    pltpu.sync_copy(pos_hbm.at[pl.ds(wid * QBLK, QBLK)], pos_v)

    def p1(i, _):
        blk = i // 8
        col = (i % 8) * L
        px = pos_v[blk, pl.ds(col, L)]
        py = pos_v[blk, pl.ds(128 + col, L)]
        ix = px.astype(jnp.int32)
        iy = py.astype(jnp.int32)
        nmidx[i // GPC, pl.ds((i % GPC) * L, L)] = ix * GRID + iy
        qx[pl.ds(i * L, L)] = ix.astype(jnp.float32)
        qy[pl.ds(i * L, L)] = iy.astype(jnp.float32)
        return 0

    lax.fori_loop(0, GPC, p1, 0)
    for k in range(K):
        pltpu.async_copy(nm_hbms[k].at[nmidx.at[0]], nidx.at[0, k], sem_nm)
    lax.fori_loop(GPC, PER_W // L, p1, 0)

    def _streams(p, sem_pt, sem_emb, launch):
        for k in range(K):
            for src, dst, sem in (
                    (pts_hbm.at[pxi.at[p, k]], ptx.at[p, k], sem_pt),
                    (pts_hbm.at[pyi.at[p, k]], pty.at[p, k], sem_pt),
                    (emb_hbm.at[nidx.at[p, k]],
                     emb_rows.at[p, pl.ds(k * CHUNK, CHUNK)], sem_emb)):
                if launch:
                    pltpu.async_copy(src, dst, sem)
                else:
                    pltpu.make_async_copy(src, dst, sem).wait()

    def fire_big(par):
        @pl.when(par == 0)
        def _():
            _streams(0, sem_pt0, sem_emb0, True)

        @pl.when(par == 1)
        def _():
            _streams(1, sem_pt1, sem_emb1, True)

    def wait_big(par):
        @pl.when(par == 0)
        def _():
            _streams(0, sem_pt0, sem_emb0, False)

        @pl.when(par == 1)
        def _():
            _streams(1, sem_pt1, sem_emb1, False)

    def compute(c, par):
        wait_big(par)

        def wstep(o, _):
            qs = pl.ds(c * CHUNK + o * L, L)
            qxv = qx[qs]
            qyv = qy[qs]
            ws = []
            for k in range(K):
                s = pl.ds(o * L, L)
                dx = ptx[par, k, s] - qxv
                dy = pty[par, k, s] - qyv
                d2 = dx * dx + dy * dy
                dist = d2 * _rsqrt(d2)
                ws.append(1.0 / (dist + 1e-6))
            inv = 1.0 / (ws[0] + ws[1] + ws[2] + ws[3])
            for k in range(K):
                wn[k, pl.ds(o * L, L)] = ws[k] * inv
            return 0

        lax.fori_loop(0, GPC, wstep, 0)

        def comb(g, _):
            s = pl.ds(g * L, L)
            wv = [wn[k, s] for k in range(K)]
            for lane in range(L):
                q = g * L + lane
                w = [wv[k][lane] for k in range(K)]
                e = [emb_rows.at[par, k * CHUNK + q] for k in range(K)]
                for h in range(2):
                    hs = pl.ds(h * L, L)
                    acc = (w[0] * e[0][hs] + w[1] * e[1][hs]
                           + w[2] * e[2][hs] + w[3] * e[3][hs])
                    out_v[q, hs] = acc
            return 0

        lax.fori_loop(0, GPC, comb, 0)
        pltpu.sync_copy(out_v, out_hbm.at[pl.ds(base + c * CHUNK, CHUNK)])

    def chunk(c, _):
        par = c & 1

        for k in range(K):
            pltpu.make_async_copy(nm_hbms[k].at[nmidx.at[c]],
                                  nidx.at[par, k], sem_nm).wait()
        for k in range(K):
            for o in range(GPC):
                s = pl.ds(o * L, L)
                nb = nidx[par, k, s]
                xi = nb + (nb & jnp.int32(-128))
                pxi[par, k, s] = xi
                pyi[par, k, s] = xi + 128
        fire_big(par)

        @pl.when(c < NCH - 1)
        def _():
            for k in range(K):
                pltpu.async_copy(nm_hbms[k].at[nmidx.at[c + 1]],
                                 nidx.at[1 - par, k], sem_nm)

        @pl.when(c > 0)
        def _():
            compute(c - 1, 1 - par)

        return 0

    lax.fori_loop(0, NCH, chunk, 0)
    compute(NCH - 1, (NCH - 1) & 1)


@jax.jit
def _run(pos_blk, pts_flat, embeddings, nm0, nm1, nm2, nm3):
    mesh = plsc.VectorSubcoreMesh(core_axis_name="c", subcore_axis_name="s")
    f = pl.kernel(
        _body,
        out_type=jax.ShapeDtypeStruct((B, D), jnp.float32),
        mesh=mesh,
        compiler_params=pltpu.CompilerParams(
            needs_layout_passes=False, use_tc_tiling_on_sc=False),
        scratch_types=[
            pltpu.VMEM((QBLK, 256), jnp.float32),
            pltpu.VMEM((PER_W,), jnp.float32),
            pltpu.VMEM((PER_W,), jnp.float32),
            pltpu.VMEM((NCH, CHUNK), jnp.int32),
            pltpu.VMEM((2, K, CHUNK), jnp.int32),
            pltpu.VMEM((2, K, CHUNK), jnp.int32),
            pltpu.VMEM((2, K, CHUNK), jnp.int32),
            pltpu.VMEM((2, K, CHUNK), jnp.float32),
            pltpu.VMEM((2, K, CHUNK), jnp.float32),
            pltpu.VMEM((2, K * CHUNK, D), jnp.float32),
            pltpu.VMEM((K, CHUNK), jnp.float32),
            pltpu.VMEM((CHUNK, D), jnp.float32),
            pltpu.SemaphoreType.DMA,
            pltpu.SemaphoreType.DMA,
            pltpu.SemaphoreType.DMA,
            pltpu.SemaphoreType.DMA,
            pltpu.SemaphoreType.DMA,
        ],
    )
    return f(pos_blk, pts_flat, embeddings, nm0, nm1, nm2, nm3)


def kernel(position, positions, embeddings, neighbor_map):
    pos_blk = position.reshape(B // 128, 128, 2).transpose(0, 2, 1) \
                      .reshape(B // 128, 256)
    pts_pad = jnp.concatenate(
        [positions, jnp.zeros((NP_PAD - N_POINTS, 2), positions.dtype)])
    pts_flat = pts_pad.reshape(NP_PAD // 128, 128, 2).transpose(0, 2, 1) \
                      .reshape(NP_PAD * 2)
    nms = [neighbor_map[:, :, k].reshape(GRID * GRID) for k in range(K)]
    return _run(pos_blk, pts_flat, embeddings, *nms)


# speedup vs baseline: 1.0741x; 1.0741x over previous
import jax
import jax.numpy as jnp
from jax import lax
from jax.experimental import pallas as pl
from jax.experimental.pallas import tpu as pltpu
from jax.experimental.pallas import tpu_sc as plsc

N_POINTS = 1_000_000
GRID = 1024
D = 32
K = 4
B = 65536

NC = 2
NS = 16
L = 16
NW = NC * NS
PER_W = B // NW
CHUNK = 128
NCH = PER_W // CHUNK
GPC = CHUNK // L


def _rsqrt(x):
    i = lax.bitcast_convert_type(x, jnp.int32)
    i = jnp.int32(0x5F3759DF) - lax.shift_right_arithmetic(i, 1)
    y = lax.bitcast_convert_type(i, jnp.float32)
    for _ in range(3):
        y = y * (1.5 - 0.5 * x * y * y)
    return y


def _body(posx_hbm, posy_hbm, ptsx_hbm, ptsy_hbm, emb_hbm, nm_hbm, out_hbm,
          qx, qy, nmidx, nidx, ptx, pty, emb_rows, wn, out_v,
          sem_nm, sem_pt0, sem_pt1, sem_emb0, sem_emb1):
    wid = lax.axis_index("s") * NC + lax.axis_index("c")
    base = wid * PER_W

    pltpu.sync_copy(posx_hbm.at[pl.ds(base, PER_W)], qx)
    pltpu.sync_copy(posy_hbm.at[pl.ds(base, PER_W)], qy)

    def p1(i, _):
        s = pl.ds(i * L, L)
        ix = qx[s].astype(jnp.int32)
        iy = qy[s].astype(jnp.int32)
        nm0 = ix * 4096 + lax.shift_right_arithmetic(iy, 7) * 512 \
            + (iy & 127)
        c = i // GPC
        o = (i % GPC) * L
        for k in range(K):
            nmidx[k, c, pl.ds(o, L)] = nm0 + k * 128
        qx[s] = ix.astype(jnp.float32)
        qy[s] = iy.astype(jnp.float32)
        return 0

    lax.fori_loop(0, GPC, p1, 0)
    for k in range(K):
        pltpu.async_copy(nm_hbm.at[nmidx.at[k, 0]], nidx.at[0, k], sem_nm)
    lax.fori_loop(GPC, PER_W // L, p1, 0)

    def _streams(p, sem_pt, sem_emb, launch):
        for k in range(K):
            for src, dst, sem in (
                    (ptsx_hbm.at[nidx.at[p, k]], ptx.at[p, k], sem_pt),
                    (ptsy_hbm.at[nidx.at[p, k]], pty.at[p, k], sem_pt),
                    (emb_hbm.at[nidx.at[p, k]],
                     emb_rows.at[p, pl.ds(k * CHUNK, CHUNK)], sem_emb)):
                if launch:
                    pltpu.async_copy(src, dst, sem)
                else:
                    pltpu.make_async_copy(src, dst, sem).wait()

    def fire_big(par):
        @pl.when(par == 0)
        def _():
            _streams(0, sem_pt0, sem_emb0, True)

        @pl.when(par == 1)
        def _():
            _streams(1, sem_pt1, sem_emb1, True)

    def wait_big(par):
        @pl.when(par == 0)
        def _():
            _streams(0, sem_pt0, sem_emb0, False)

        @pl.when(par == 1)
        def _():
            _streams(1, sem_pt1, sem_emb1, False)

    def compute(c, par):
        wait_big(par)

        def wstep(o, _):
            qs = pl.ds(c * CHUNK + o * L, L)
            qxv = qx[qs]
            qyv = qy[qs]
            ws = []
            for k in range(K):
                s = pl.ds(o * L, L)
                dx = ptx[par, k, s] - qxv
                dy = pty[par, k, s] - qyv
                d2 = dx * dx + dy * dy
                dist = d2 * _rsqrt(d2)
                ws.append(1.0 / (dist + 1e-6))
            inv = 1.0 / (ws[0] + ws[1] + ws[2] + ws[3])
            for k in range(K):
                wn[k, pl.ds(o * L, L)] = ws[k] * inv
            return 0

        lax.fori_loop(0, GPC, wstep, 0)

        def comb(g, _):
            s = pl.ds(g * L, L)
            wv = [wn[k, s] for k in range(K)]
            for lane in range(L):
                q = g * L + lane
                w = [wv[k][lane] for k in range(K)]
                e = [emb_rows.at[par, k * CHUNK + q] for k in range(K)]
                for h in range(2):
                    hs = pl.ds(h * L, L)
                    acc = (w[0] * e[0][hs] + w[1] * e[1][hs]
                           + w[2] * e[2][hs] + w[3] * e[3][hs])
                    out_v[q, hs] = acc
            return 0

        lax.fori_loop(0, GPC, comb, 0)
        pltpu.sync_copy(out_v, out_hbm.at[pl.ds(base + c * CHUNK, CHUNK)])

    def chunk(c, _):
        par = c & 1

        for k in range(K):
            pltpu.make_async_copy(nm_hbm.at[nmidx.at[k, c]],
                                  nidx.at[par, k], sem_nm).wait()
        fire_big(par)

        @pl.when(c < NCH - 1)
        def _():
            for k in range(K):
                pltpu.async_copy(nm_hbm.at[nmidx.at[k, c + 1]],
                                 nidx.at[1 - par, k], sem_nm)

        @pl.when(c > 0)
        def _():
            compute(c - 1, 1 - par)

        return 0

    lax.fori_loop(0, NCH, chunk, 0)
    compute(NCH - 1, (NCH - 1) & 1)


@jax.jit
def _run(posx, posy, ptsx, ptsy, embeddings, nm_flat):
    mesh = plsc.VectorSubcoreMesh(core_axis_name="c", subcore_axis_name="s")
    f = pl.kernel(
        _body,
        out_type=jax.ShapeDtypeStruct((B, D), jnp.float32),
        mesh=mesh,
        compiler_params=pltpu.CompilerParams(
            needs_layout_passes=False, use_tc_tiling_on_sc=False),
        scratch_types=[
            pltpu.VMEM((PER_W,), jnp.float32),
            pltpu.VMEM((PER_W,), jnp.float32),
            pltpu.VMEM((K, NCH, CHUNK), jnp.int32),
            pltpu.VMEM((2, K, CHUNK), jnp.int32),
            pltpu.VMEM((2, K, CHUNK), jnp.float32),
            pltpu.VMEM((2, K, CHUNK), jnp.float32),
            pltpu.VMEM((2, K * CHUNK, D), jnp.float32),
            pltpu.VMEM((K, CHUNK), jnp.float32),
            pltpu.VMEM((CHUNK, D), jnp.float32),
            pltpu.SemaphoreType.DMA,
            pltpu.SemaphoreType.DMA,
            pltpu.SemaphoreType.DMA,
            pltpu.SemaphoreType.DMA,
            pltpu.SemaphoreType.DMA,
        ],
    )
    return f(posx, posy, ptsx, ptsy, embeddings, nm_flat)


def kernel(position, positions, embeddings, neighbor_map):
    nm_flat = neighbor_map.reshape(GRID, 8, 128, K).transpose(0, 1, 3, 2) \
                          .reshape(GRID * GRID * K)
    return _run(position[:, 0], position[:, 1],
                positions[:, 0], positions[:, 1],
                embeddings, nm_flat)
